# baseline (speedup 1.0000x reference)
import jax
import jax.numpy as jnp
from jax import lax
from jax.experimental import pallas as pl
from jax.experimental.pallas import tpu as pltpu
from jax.experimental.pallas import tpu_sc as plsc

N0 = 4096
NUM_OMICS = 3
NN = N0 * NUM_OMICS
EE = 393216
RR = 6
DD = 128
SQK_STRIDE = 16

NUM_SC = 2
NUM_TILES = 16
NW = NUM_SC * NUM_TILES
TILE_E = EE // NW

E1_BLK = 1024
E2_BLK = 128
NPT = NN // NUM_TILES


def _elu(x):
    return jnp.where(x > 0, x, jnp.exp(jnp.minimum(x, 0.0)) - 1.0)


def _proj3_body(
    x0_ref, x1_ref, x2_ref, pw0_ref, pw1_ref, pw2_ref, pb_ref,
    s1w_ref, s1b_ref, xh_ref, sl_ref,
):
    for i, (x_ref, pw_ref) in enumerate(
        ((x0_ref, pw0_ref), (x1_ref, pw1_ref), (x2_ref, pw2_ref))
    ):
        p = jnp.dot(x_ref[...], pw_ref[...], preferred_element_type=jnp.float32)
        xh = _elu(p + pb_ref[i])
        xh_ref[i] = xh
        sl_ref[i] = (
            jnp.dot(xh, s1w_ref[i], preferred_element_type=jnp.float32)
            + s1b_ref[i]
        )


def _proj3(x0, x1, x2, pw0, pw1, pw2, pb_stack, s1w, s1b):
    bn = 1024
    return pl.pallas_call(
        _proj3_body,
        grid=(N0 // bn,),
        in_specs=[
            pl.BlockSpec((bn, x0.shape[1]), lambda b: (b, 0)),
            pl.BlockSpec((bn, x1.shape[1]), lambda b: (b, 0)),
            pl.BlockSpec((bn, x2.shape[1]), lambda b: (b, 0)),
            pl.BlockSpec((x0.shape[1], DD), lambda b: (0, 0)),
            pl.BlockSpec((x1.shape[1], DD), lambda b: (0, 0)),
            pl.BlockSpec((x2.shape[1], DD), lambda b: (0, 0)),
            pl.BlockSpec((NUM_OMICS, 1, DD), lambda b: (0, 0, 0)),
            pl.BlockSpec((NUM_OMICS, DD, DD), lambda b: (0, 0, 0)),
            pl.BlockSpec((NUM_OMICS, 1, DD), lambda b: (0, 0, 0)),
        ],
        out_specs=[
            pl.BlockSpec((NUM_OMICS, bn, DD), lambda b: (0, b, 0)),
            pl.BlockSpec((NUM_OMICS, bn, DD), lambda b: (0, b, 0)),
        ],
        out_shape=[
            jax.ShapeDtypeStruct((NUM_OMICS, N0, DD), jnp.float32),
            jax.ShapeDtypeStruct((NUM_OMICS, N0, DD), jnp.float32),
        ],
    )(x0, x1, x2, pw0, pw1, pw2, pb_stack, s1w, s1b.reshape(NUM_OMICS, 1, DD))


def _hall_body(x_ref, w_ref, q_ref, k_ref, h_ref, sq_ref, sk_ref):
    x = x_ref[...]
    for r in range(RR):
        h = jnp.dot(x, w_ref[r], preferred_element_type=jnp.float32)
        h_ref[r] = h
        sq_ref[r] = jnp.sum(h * q_ref[r], axis=1, keepdims=True)
        sk_ref[r] = jnp.sum(h * k_ref[r], axis=1, keepdims=True)


def _hall_scores(x, w, q, k):
    bn = 512
    return pl.pallas_call(
        _hall_body,
        grid=(NN // bn,),
        in_specs=[
            pl.BlockSpec((bn, DD), lambda b: (b, 0)),
            pl.BlockSpec((RR, DD, DD), lambda b: (0, 0, 0)),
            pl.BlockSpec((RR, 1, DD), lambda b: (0, 0, 0)),
            pl.BlockSpec((RR, 1, DD), lambda b: (0, 0, 0)),
        ],
        out_specs=[
            pl.BlockSpec((RR, bn, DD), lambda b: (0, b, 0)),
            pl.BlockSpec((RR, bn, 1), lambda b: (0, b, 0)),
            pl.BlockSpec((RR, bn, 1), lambda b: (0, b, 0)),
        ],
        out_shape=[
            jax.ShapeDtypeStruct((RR, NN, DD), jnp.float32),
            jax.ShapeDtypeStruct((RR, NN, 1), jnp.float32),
            jax.ShapeDtypeStruct((RR, NN, 1), jnp.float32),
        ],
    )(x, w, q.reshape(RR, 1, DD), k.reshape(RR, 1, DD))


def _hallc_body(
    op_ref, sp_ref, dp_ref, b_ref, sl_ref, w_ref, q_ref, k_ref,
    h_ref, sq_ref, sk_ref,
):
    s = sp_ref[0] + sp_ref[1]
    deg = dp_ref[0] + dp_ref[1]
    wn = deg / (s + 1e-16)
    x = _elu(wn * (op_ref[0] + op_ref[1]) + b_ref[...] + sl_ref[...])
    for r in range(RR):
        h = jnp.dot(x, w_ref[r], preferred_element_type=jnp.float32)
        h_ref[r] = h
        sq_ref[r] = jnp.sum(h * q_ref[r], axis=1, keepdims=True)
        sk_ref[r] = jnp.sum(h * k_ref[r], axis=1, keepdims=True)


def _hall_scores_combined(outp, s_part, deg_part, b, sl, w, q, k):
    bn = 512
    return pl.pallas_call(
        _hallc_body,
        grid=(NN // bn,),
        in_specs=[
            pl.BlockSpec((2, bn, DD), lambda b_: (0, b_, 0)),
            pl.BlockSpec((2, bn, 1), lambda b_: (0, b_, 0)),
            pl.BlockSpec((2, bn, 1), lambda b_: (0, b_, 0)),
            pl.BlockSpec((1, DD), lambda b_: (0, 0)),
            pl.BlockSpec((bn, DD), lambda b_: (b_, 0)),
            pl.BlockSpec((RR, DD, DD), lambda b_: (0, 0, 0)),
            pl.BlockSpec((RR, 1, DD), lambda b_: (0, 0, 0)),
            pl.BlockSpec((RR, 1, DD), lambda b_: (0, 0, 0)),
        ],
        out_specs=[
            pl.BlockSpec((RR, bn, DD), lambda b_: (0, b_, 0)),
            pl.BlockSpec((RR, bn, 1), lambda b_: (0, b_, 0)),
            pl.BlockSpec((RR, bn, 1), lambda b_: (0, b_, 0)),
        ],
        out_shape=[
            jax.ShapeDtypeStruct((RR, NN, DD), jnp.float32),
            jax.ShapeDtypeStruct((RR, NN, 1), jnp.float32),
            jax.ShapeDtypeStruct((RR, NN, 1), jnp.float32),
        ],
    )(
        outp,
        s_part.reshape(2, NN, 1),
        deg_part.reshape(2, NN, 1),
        b.reshape(1, DD),
        sl,
        w,
        q.reshape(RR, 1, DD),
        k.reshape(RR, 1, DD),
    )


def _combine_body(op_ref, sp_ref, dp_ref, b_ref, sl_ref, out_ref):
    s = sp_ref[0] + sp_ref[1]
    deg = dp_ref[0] + dp_ref[1]
    w = deg / (s + 1e-16)
    h = w * (op_ref[0] + op_ref[1]) + b_ref[...] + sl_ref[...]
    out_ref[...] = _elu(h)


def _combine(outp, s_part, deg_part, b, sl):
    bn = 256
    return pl.pallas_call(
        _combine_body,
        grid=(NN // bn,),
        in_specs=[
            pl.BlockSpec((2, bn, DD), lambda b_: (0, b_, 0)),
            pl.BlockSpec((2, bn, 1), lambda b_: (0, b_, 0)),
            pl.BlockSpec((2, bn, 1), lambda b_: (0, b_, 0)),
            pl.BlockSpec((1, DD), lambda b_: (0, 0)),
            pl.BlockSpec((bn, DD), lambda b_: (b_, 0)),
        ],
        out_specs=pl.BlockSpec((bn, DD), lambda b_: (b_, 0)),
        out_shape=jax.ShapeDtypeStruct((NN, DD), jnp.float32),
    )(
        outp,
        s_part.reshape(2, NN, 1),
        deg_part.reshape(2, NN, 1),
        b.reshape(1, DD),
        sl,
    )


def _head_body(
    op_ref, sp_ref, dp_ref, b_ref, sl_ref, l1w_ref, l1b_ref, l2w_ref, l2b_ref, y_ref
):
    s = sp_ref[0] + sp_ref[1]
    deg = dp_ref[0] + dp_ref[1]
    w = deg / (s + 1e-16)
    h = _elu(w * (op_ref[0] + op_ref[1]) + b_ref[...] + sl_ref[...])
    z = _elu(jnp.dot(h, l1w_ref[...], preferred_element_type=jnp.float32) + l1b_ref[...])
    y_ref[...] = (
        jnp.dot(z, l2w_ref[...], preferred_element_type=jnp.float32) + l2b_ref[...]
    )


def _head(outp, s_part, deg_part, b, sl, l1w, l1b, l2w_pad, l2b_pad):
    bn = 1024
    return pl.pallas_call(
        _head_body,
        grid=(N0 // bn,),
        in_specs=[
            pl.BlockSpec((2, bn, DD), lambda b_: (0, b_, 0)),
            pl.BlockSpec((2, bn, 1), lambda b_: (0, b_, 0)),
            pl.BlockSpec((2, bn, 1), lambda b_: (0, b_, 0)),
            pl.BlockSpec((1, DD), lambda b_: (0, 0)),
            pl.BlockSpec((bn, DD), lambda b_: (b_, 0)),
            pl.BlockSpec((DD, DD), lambda b_: (0, 0)),
            pl.BlockSpec((1, DD), lambda b_: (0, 0)),
            pl.BlockSpec((DD, DD), lambda b_: (0, 0)),
            pl.BlockSpec((1, DD), lambda b_: (0, 0)),
        ],
        out_specs=pl.BlockSpec((bn, DD), lambda b_: (b_, 0)),
        out_shape=jax.ShapeDtypeStruct((N0, DD), jnp.float32),
    )(
        outp,
        s_part.reshape(2, NN, 1),
        deg_part.reshape(2, NN, 1),
        b.reshape(1, DD),
        sl,
        l1w,
        l1b.reshape(1, DD),
        l2w_pad,
        l2b_pad.reshape(1, DD),
    )


EB = 768
RB = 32
NIR = EB // RB
NSB = TILE_E // EB
NRING = 4


def _zero_fill(buf, nwords):
    z = jnp.zeros((16,), jnp.float32)

    def body(i, _):
        buf[pl.ds(i * 16, 16)] = z
        return 0

    lax.fori_loop(0, nwords // 16, body, 0)


def _edge_body(
    sq_hbm, sk_hbm, hall_hbm, src_hbm, dst_hbm, et_hbm,
    s_out, deg_out, out_hbm,
    src_v, dst_v, et_v, idxd_v, idxs_v, dstw_v,
    val_s, e_v, ones_v, r0_v, r1_v, r2_v, r3_v,
    s_sh, deg_sh, out_sh,
    sem_q, sem_sd, sg0, sg1, sg2, sg3, ss0, ss1, ss2, ss3, sem_z,
):
    cid = lax.axis_index("c")
    sid = lax.axis_index("s")
    base0 = (cid * NUM_TILES + sid) * TILE_E
    rows = (r0_v, r1_v, r2_v, r3_v)
    sgs = (sg0, sg1, sg2, sg3)
    sss = (ss0, ss1, ss2, ss3)

    def zf(i, _):
        r0_v[i // 8, pl.ds((i % 8) * 16, 16)] = jnp.zeros((16,), jnp.float32)
        return 0

    lax.fori_loop(0, RB * 8, zf, 0)
    _zero_fill(e_v, EB)

    def fo(i, _):
        ones_v[pl.ds(i * 16, 16)] = jnp.ones((16,), jnp.float32)
        return 0

    lax.fori_loop(0, RB // 16, fo, 0)

    zc = [
        pltpu.async_copy(
            r0_v, out_sh.at[pl.ds(sid * NPT + i * RB, RB), :], sem_z
        )
        for i in range(NPT // RB)
    ]
    for cp in zc:
        cp.wait()
    pltpu.sync_copy(e_v.at[pl.ds(0, NPT)], s_sh.at[pl.ds(sid * NPT, NPT)])
    pltpu.sync_copy(e_v.at[pl.ds(0, NPT)], deg_sh.at[pl.ds(sid * NPT, NPT)])
    plsc.subcore_barrier()

    def _scale(rbuf, eoff):
        def sc(g, _):
            cf16 = e_v[pl.ds(eoff + g * 16, 16)]
            for jj in range(16):
                cf = cf16[jj]
                for j in range(DD // 16):
                    sl2 = pl.ds(j * 16, 16)
                    rbuf[g * 16 + jj, sl2] = rbuf[g * 16 + jj, sl2] * cf
            return 0

        lax.fori_loop(0, RB // 16, sc, 0)

    def _g_fire(r, j):
        return pltpu.async_copy(hall_hbm.at[idxs_v.at[r]], rows[j], sgs[j])

    def _g_wait(r, j):
        pltpu.make_async_copy(hall_hbm.at[idxs_v.at[r]], rows[j], sgs[j]).wait()

    def _s_fire(r, j):
        return pltpu.async_copy(rows[j], out_sh.at[dstw_v.at[r]], sss[j], add=True)

    def _s_wait(r, j):
        pltpu.make_async_copy(rows[j], out_sh.at[dstw_v.at[r]], sss[j]).wait()

    def sb_body(sb, _):
        base = base0 + sb * EB
        pltpu.sync_copy(src_hbm.at[pl.ds(base, EB)], src_v)
        pltpu.sync_copy(dst_hbm.at[pl.ds(base, EB)], dst_v)
        pltpu.sync_copy(et_hbm.at[pl.ds(base, EB)], et_v)

        def ib(i, _):
            sl = pl.ds(i * 16, 16)
            d16 = dst_v[sl]
            enn = et_v[sl] * NN
            r = i // 2
            o = (i % 2) * 16
            idxd_v[r, pl.ds(o, 16)] = enn + d16
            idxs_v[r, pl.ds(o, 16)] = enn + src_v[sl]
            dstw_v[r, pl.ds(o, 16)] = d16
            return 0

        lax.fori_loop(0, EB // 16, ib, 0)

        g_pend0 = _g_fire(0, 0)
        g_pend1 = _g_fire(1, 1)
        qc = []
        for c in range(NIR):
            qc.append(pltpu.async_copy(
                sq_hbm.at[idxd_v.at[c]], e_v.at[pl.ds(c * RB, RB)], sem_q))
            qc.append(pltpu.async_copy(
                sk_hbm.at[idxs_v.at[c]], val_s.at[pl.ds(c * RB, RB)], sem_q))
        for cp in qc:
            cp.wait()

        def lg(i, _):
            sl = pl.ds(i * 16, 16)
            l = e_v[sl] + val_s[sl]
            l = jnp.maximum(l, 0.2 * l)
            e_v[sl] = jnp.exp(l)
            return 0

        lax.fori_loop(0, EB // 16, lg, 0)

        sd = []
        for c in range(NIR):
            sl = pl.ds(c * RB, RB)
            sd.append(pltpu.async_copy(
                e_v.at[sl], s_sh.at[dstw_v.at[c]], sem_sd, add=True))
            sd.append(pltpu.async_copy(
                ones_v, deg_sh.at[dstw_v.at[c]], sem_sd, add=True))

        def quad_body(t, _):
            for j in range(NRING):
                r = t * NRING + j
                _g_wait(r, j)
                j2 = (j + 2) % NRING

                @pl.when(r + 2 < NIR)
                def _():
                    @pl.when(r >= 2)
                    def _():
                        _s_wait(r - 2, j2)
                    _g_fire(r + 2, j2)

                _scale(rows[j], r * RB)
                _s_fire(r, j)
            return 0

        lax.fori_loop(0, NIR // NRING, quad_body, 0)

        _s_wait(NIR - 2, (NIR - 2) % NRING)
        _s_wait(NIR - 1, (NIR - 1) % NRING)
        for cp in sd:
            cp.wait()
        return 0

    lax.fori_loop(0, NSB, sb_body, 0)
    plsc.subcore_barrier()

    sl = pl.ds(sid * NPT, NPT)
    pltpu.sync_copy(s_sh.at[sl], s_out.at[cid, sl])
    pltpu.sync_copy(deg_sh.at[sl], deg_out.at[cid, sl])
    pltpu.sync_copy(out_sh.at[sl, :], out_hbm.at[cid, sl, :])


def _make_edge():
  return pl.kernel(
    _edge_body,
    out_type=[
        jax.ShapeDtypeStruct((NUM_SC, NN), jnp.float32),
        jax.ShapeDtypeStruct((NUM_SC, NN), jnp.float32),
        jax.ShapeDtypeStruct((NUM_SC, NN, DD), jnp.float32),
    ],
    mesh=plsc.VectorSubcoreMesh(core_axis_name="c", subcore_axis_name="s"),
    scratch_types=[
        pltpu.VMEM((EB,), jnp.int32),
        pltpu.VMEM((EB,), jnp.int32),
        pltpu.VMEM((EB,), jnp.int32),
        pltpu.VMEM((NIR, RB), jnp.int32),
        pltpu.VMEM((NIR, RB), jnp.int32),
        pltpu.VMEM((NIR, RB), jnp.int32),
        pltpu.VMEM((EB,), jnp.float32),
        pltpu.VMEM((EB,), jnp.float32),
        pltpu.VMEM((RB,), jnp.float32),
        pltpu.VMEM((RB, DD), jnp.float32),
        pltpu.VMEM((RB, DD), jnp.float32),
        pltpu.VMEM((RB, DD), jnp.float32),
        pltpu.VMEM((RB, DD), jnp.float32),
        pltpu.VMEM_SHARED((NN,), jnp.float32),
        pltpu.VMEM_SHARED((NN,), jnp.float32),
        pltpu.VMEM_SHARED((NN, DD), jnp.float32),
        pltpu.SemaphoreType.DMA,
        pltpu.SemaphoreType.DMA,
        pltpu.SemaphoreType.DMA,
        pltpu.SemaphoreType.DMA,
        pltpu.SemaphoreType.DMA,
        pltpu.SemaphoreType.DMA,
        pltpu.SemaphoreType.DMA,
        pltpu.SemaphoreType.DMA,
        pltpu.SemaphoreType.DMA,
        pltpu.SemaphoreType.DMA,
        pltpu.SemaphoreType.DMA,
    ],
  )


def kernel(
    x0, x1, x2, Pw0, Pb0, Pw1, Pb1, Pw2, Pb2,
    W1, q1, k1, b1, W2, q2, k2, b2,
    S1w, S1b, L1w, L1b, L2w, L2b, edge_index, edge_type,
):
    src = edge_index[0]
    dst = edge_index[1]
    et = edge_type

    l2w_pad = jnp.pad(L2w, ((0, 0), (0, DD - L2w.shape[1])))
    l2b_pad = jnp.pad(L2b, (0, DD - L2b.shape[0]))

    pb_stack = jnp.stack([Pb0, Pb1, Pb2]).reshape(NUM_OMICS, 1, DD)
    xh3, sl3 = _proj3(x0, x1, x2, Pw0, Pw1, Pw2, pb_stack, S1w, S1b)
    xh = xh3.reshape(NN, DD)
    sl = sl3.reshape(NN, DD)

    edge_call = _make_edge()

    hall1, sq1, sk1 = _hall_scores(xh, W1, q1, k1)
    s1p, degp, op1 = edge_call(
        sq1.reshape(RR * NN), sk1.reshape(RR * NN),
        hall1.reshape(RR * NN, DD), src, dst, et,
    )
    hall2, sq2, sk2 = _hall_scores_combined(op1, s1p, degp, b1, sl, W2, q2, k2)
    s2p, _deg2, op2 = edge_call(
        sq2.reshape(RR * NN), sk2.reshape(RR * NN),
        hall2.reshape(RR * NN, DD), src, dst, et,
    )

    y = _head(op2, s2p, degp, b2, sl, L1w, L1b, l2w_pad, l2b_pad)
    return y[:, : L2w.shape[1]]


# speedup vs baseline: 1.0180x; 1.0180x over previous
import jax
import jax.numpy as jnp
from jax import lax
from jax.experimental import pallas as pl
from jax.experimental.pallas import tpu as pltpu
from jax.experimental.pallas import tpu_sc as plsc

N0 = 4096
NUM_OMICS = 3
NN = N0 * NUM_OMICS
EE = 393216
RR = 6
DD = 128
SQK_STRIDE = 16

NUM_SC = 2
NUM_TILES = 16
NW = NUM_SC * NUM_TILES
TILE_E = EE // NW

E1_BLK = 1024
E2_BLK = 128
NPT = NN // NUM_TILES


def _elu(x):
    return jnp.where(x > 0, x, jnp.exp(jnp.minimum(x, 0.0)) - 1.0)


def _proj3_body(
    x0_ref, x1_ref, x2_ref, pw0_ref, pw1_ref, pw2_ref, pb_ref,
    s1w_ref, s1b_ref, xh_ref, sl_ref,
):
    for i, (x_ref, pw_ref) in enumerate(
        ((x0_ref, pw0_ref), (x1_ref, pw1_ref), (x2_ref, pw2_ref))
    ):
        p = jnp.dot(x_ref[...], pw_ref[...], preferred_element_type=jnp.float32)
        xh = _elu(p + pb_ref[i])
        xh_ref[i] = xh
        sl_ref[i] = (
            jnp.dot(xh, s1w_ref[i], preferred_element_type=jnp.float32)
            + s1b_ref[i]
        )


def _proj3(x0, x1, x2, pw0, pw1, pw2, pb_stack, s1w, s1b):
    bn = 1024
    return pl.pallas_call(
        _proj3_body,
        grid=(N0 // bn,),
        in_specs=[
            pl.BlockSpec((bn, x0.shape[1]), lambda b: (b, 0)),
            pl.BlockSpec((bn, x1.shape[1]), lambda b: (b, 0)),
            pl.BlockSpec((bn, x2.shape[1]), lambda b: (b, 0)),
            pl.BlockSpec((x0.shape[1], DD), lambda b: (0, 0)),
            pl.BlockSpec((x1.shape[1], DD), lambda b: (0, 0)),
            pl.BlockSpec((x2.shape[1], DD), lambda b: (0, 0)),
            pl.BlockSpec((NUM_OMICS, 1, DD), lambda b: (0, 0, 0)),
            pl.BlockSpec((NUM_OMICS, DD, DD), lambda b: (0, 0, 0)),
            pl.BlockSpec((NUM_OMICS, 1, DD), lambda b: (0, 0, 0)),
        ],
        out_specs=[
            pl.BlockSpec((NUM_OMICS, bn, DD), lambda b: (0, b, 0)),
            pl.BlockSpec((NUM_OMICS, bn, DD), lambda b: (0, b, 0)),
        ],
        out_shape=[
            jax.ShapeDtypeStruct((NUM_OMICS, N0, DD), jnp.float32),
            jax.ShapeDtypeStruct((NUM_OMICS, N0, DD), jnp.float32),
        ],
    )(x0, x1, x2, pw0, pw1, pw2, pb_stack, s1w, s1b.reshape(NUM_OMICS, 1, DD))


def _hall_body(x_ref, w_ref, q_ref, k_ref, h_ref, sq_ref, sk_ref):
    x = x_ref[...]
    for r in range(RR):
        h = jnp.dot(x, w_ref[r], preferred_element_type=jnp.float32)
        h_ref[r] = h
        sq_ref[r] = jnp.sum(h * q_ref[r], axis=1, keepdims=True)
        sk_ref[r] = jnp.sum(h * k_ref[r], axis=1, keepdims=True)


def _hall_scores(x, w, q, k):
    bn = 1024
    return pl.pallas_call(
        _hall_body,
        grid=(NN // bn,),
        in_specs=[
            pl.BlockSpec((bn, DD), lambda b: (b, 0)),
            pl.BlockSpec((RR, DD, DD), lambda b: (0, 0, 0)),
            pl.BlockSpec((RR, 1, DD), lambda b: (0, 0, 0)),
            pl.BlockSpec((RR, 1, DD), lambda b: (0, 0, 0)),
        ],
        out_specs=[
            pl.BlockSpec((RR, bn, DD), lambda b: (0, b, 0)),
            pl.BlockSpec((RR, bn, 1), lambda b: (0, b, 0)),
            pl.BlockSpec((RR, bn, 1), lambda b: (0, b, 0)),
        ],
        out_shape=[
            jax.ShapeDtypeStruct((RR, NN, DD), jnp.float32),
            jax.ShapeDtypeStruct((RR, NN, 1), jnp.float32),
            jax.ShapeDtypeStruct((RR, NN, 1), jnp.float32),
        ],
    )(x, w, q.reshape(RR, 1, DD), k.reshape(RR, 1, DD))


def _hallc_body(
    op_ref, sp_ref, dp_ref, b_ref, sl_ref, w_ref, q_ref, k_ref,
    h_ref, sq_ref, sk_ref,
):
    s = sp_ref[0] + sp_ref[1]
    deg = dp_ref[0] + dp_ref[1]
    wn = deg / (s + 1e-16)
    x = _elu(wn * (op_ref[0] + op_ref[1]) + b_ref[...] + sl_ref[...])
    for r in range(RR):
        h = jnp.dot(x, w_ref[r], preferred_element_type=jnp.float32)
        h_ref[r] = h
        sq_ref[r] = jnp.sum(h * q_ref[r], axis=1, keepdims=True)
        sk_ref[r] = jnp.sum(h * k_ref[r], axis=1, keepdims=True)


def _hall_scores_combined(outp, s_part, deg_part, b, sl, w, q, k):
    bn = 1024
    return pl.pallas_call(
        _hallc_body,
        grid=(NN // bn,),
        in_specs=[
            pl.BlockSpec((2, bn, DD), lambda b_: (0, b_, 0)),
            pl.BlockSpec((2, bn, 1), lambda b_: (0, b_, 0)),
            pl.BlockSpec((2, bn, 1), lambda b_: (0, b_, 0)),
            pl.BlockSpec((1, DD), lambda b_: (0, 0)),
            pl.BlockSpec((bn, DD), lambda b_: (b_, 0)),
            pl.BlockSpec((RR, DD, DD), lambda b_: (0, 0, 0)),
            pl.BlockSpec((RR, 1, DD), lambda b_: (0, 0, 0)),
            pl.BlockSpec((RR, 1, DD), lambda b_: (0, 0, 0)),
        ],
        out_specs=[
            pl.BlockSpec((RR, bn, DD), lambda b_: (0, b_, 0)),
            pl.BlockSpec((RR, bn, 1), lambda b_: (0, b_, 0)),
            pl.BlockSpec((RR, bn, 1), lambda b_: (0, b_, 0)),
        ],
        out_shape=[
            jax.ShapeDtypeStruct((RR, NN, DD), jnp.float32),
            jax.ShapeDtypeStruct((RR, NN, 1), jnp.float32),
            jax.ShapeDtypeStruct((RR, NN, 1), jnp.float32),
        ],
    )(
        outp,
        s_part.reshape(2, NN, 1),
        deg_part.reshape(2, NN, 1),
        b.reshape(1, DD),
        sl,
        w,
        q.reshape(RR, 1, DD),
        k.reshape(RR, 1, DD),
    )


def _combine_body(op_ref, sp_ref, dp_ref, b_ref, sl_ref, out_ref):
    s = sp_ref[0] + sp_ref[1]
    deg = dp_ref[0] + dp_ref[1]
    w = deg / (s + 1e-16)
    h = w * (op_ref[0] + op_ref[1]) + b_ref[...] + sl_ref[...]
    out_ref[...] = _elu(h)


def _combine(outp, s_part, deg_part, b, sl):
    bn = 256
    return pl.pallas_call(
        _combine_body,
        grid=(NN // bn,),
        in_specs=[
            pl.BlockSpec((2, bn, DD), lambda b_: (0, b_, 0)),
            pl.BlockSpec((2, bn, 1), lambda b_: (0, b_, 0)),
            pl.BlockSpec((2, bn, 1), lambda b_: (0, b_, 0)),
            pl.BlockSpec((1, DD), lambda b_: (0, 0)),
            pl.BlockSpec((bn, DD), lambda b_: (b_, 0)),
        ],
        out_specs=pl.BlockSpec((bn, DD), lambda b_: (b_, 0)),
        out_shape=jax.ShapeDtypeStruct((NN, DD), jnp.float32),
    )(
        outp,
        s_part.reshape(2, NN, 1),
        deg_part.reshape(2, NN, 1),
        b.reshape(1, DD),
        sl,
    )


def _head_body(
    op_ref, sp_ref, dp_ref, b_ref, sl_ref, l1w_ref, l1b_ref, l2w_ref, l2b_ref, y_ref
):
    s = sp_ref[0] + sp_ref[1]
    deg = dp_ref[0] + dp_ref[1]
    w = deg / (s + 1e-16)
    h = _elu(w * (op_ref[0] + op_ref[1]) + b_ref[...] + sl_ref[...])
    z = _elu(jnp.dot(h, l1w_ref[...], preferred_element_type=jnp.float32) + l1b_ref[...])
    y_ref[...] = (
        jnp.dot(z, l2w_ref[...], preferred_element_type=jnp.float32) + l2b_ref[...]
    )


def _head(outp, s_part, deg_part, b, sl, l1w, l1b, l2w_pad, l2b_pad):
    bn = 1024
    return pl.pallas_call(
        _head_body,
        grid=(N0 // bn,),
        in_specs=[
            pl.BlockSpec((2, bn, DD), lambda b_: (0, b_, 0)),
            pl.BlockSpec((2, bn, 1), lambda b_: (0, b_, 0)),
            pl.BlockSpec((2, bn, 1), lambda b_: (0, b_, 0)),
            pl.BlockSpec((1, DD), lambda b_: (0, 0)),
            pl.BlockSpec((bn, DD), lambda b_: (b_, 0)),
            pl.BlockSpec((DD, DD), lambda b_: (0, 0)),
            pl.BlockSpec((1, DD), lambda b_: (0, 0)),
            pl.BlockSpec((DD, DD), lambda b_: (0, 0)),
            pl.BlockSpec((1, DD), lambda b_: (0, 0)),
        ],
        out_specs=pl.BlockSpec((bn, DD), lambda b_: (b_, 0)),
        out_shape=jax.ShapeDtypeStruct((N0, DD), jnp.float32),
    )(
        outp,
        s_part.reshape(2, NN, 1),
        deg_part.reshape(2, NN, 1),
        b.reshape(1, DD),
        sl,
        l1w,
        l1b.reshape(1, DD),
        l2w_pad,
        l2b_pad.reshape(1, DD),
    )


EB = 768
RB = 32
NIR = EB // RB
NSB = TILE_E // EB
NRING = 4


def _zero_fill(buf, nwords):
    z = jnp.zeros((16,), jnp.float32)

    def body(i, _):
        buf[pl.ds(i * 16, 16)] = z
        return 0

    lax.fori_loop(0, nwords // 16, body, 0)


def _edge_body(
    with_deg,
    sq_hbm, sk_hbm, hall_hbm, src_hbm, dst_hbm, et_hbm,
    s_out, deg_out, out_hbm,
    src_v, dst_v, et_v, idxd_v, idxs_v, dstw_v,
    val_s, e_v, ones_v, r0_v, r1_v, r2_v, r3_v,
    s_sh, deg_sh, out_sh,
    sem_q, sem_sd, sg0, sg1, sg2, sg3, ss0, ss1, ss2, ss3, sem_z,
):
    cid = lax.axis_index("c")
    sid = lax.axis_index("s")
    base0 = (cid * NUM_TILES + sid) * TILE_E
    rows = (r0_v, r1_v, r2_v, r3_v)
    sgs = (sg0, sg1, sg2, sg3)
    sss = (ss0, ss1, ss2, ss3)

    def zf(i, _):
        r0_v[i // 8, pl.ds((i % 8) * 16, 16)] = jnp.zeros((16,), jnp.float32)
        return 0

    lax.fori_loop(0, RB * 8, zf, 0)
    _zero_fill(e_v, EB)

    if with_deg:
        def fo(i, _):
            ones_v[pl.ds(i * 16, 16)] = jnp.ones((16,), jnp.float32)
            return 0

        lax.fori_loop(0, RB // 16, fo, 0)

    zc = [
        pltpu.async_copy(
            r0_v, out_sh.at[pl.ds(sid * NPT + i * RB, RB), :], sem_z
        )
        for i in range(NPT // RB)
    ]
    for cp in zc:
        cp.wait()
    pltpu.sync_copy(e_v.at[pl.ds(0, NPT)], s_sh.at[pl.ds(sid * NPT, NPT)])
    if with_deg:
        pltpu.sync_copy(e_v.at[pl.ds(0, NPT)], deg_sh.at[pl.ds(sid * NPT, NPT)])
    plsc.subcore_barrier()

    def _scale(rbuf, eoff):
        def sc(g, _):
            cf16 = e_v[pl.ds(eoff + g * 16, 16)]
            for jj in range(16):
                cf = cf16[jj]
                for j in range(DD // 16):
                    sl2 = pl.ds(j * 16, 16)
                    rbuf[g * 16 + jj, sl2] = rbuf[g * 16 + jj, sl2] * cf
            return 0

        lax.fori_loop(0, RB // 16, sc, 0)

    def _g_fire(r, j):
        return pltpu.async_copy(hall_hbm.at[idxs_v.at[r]], rows[j], sgs[j])

    def _g_wait(r, j):
        pltpu.make_async_copy(hall_hbm.at[idxs_v.at[r]], rows[j], sgs[j]).wait()

    def _s_fire(r, j):
        return pltpu.async_copy(rows[j], out_sh.at[dstw_v.at[r]], sss[j], add=True)

    def _s_wait(r, j):
        pltpu.make_async_copy(rows[j], out_sh.at[dstw_v.at[r]], sss[j]).wait()

    def sb_body(sb, _):
        base = base0 + sb * EB
        pltpu.sync_copy(src_hbm.at[pl.ds(base, EB)], src_v)
        pltpu.sync_copy(dst_hbm.at[pl.ds(base, EB)], dst_v)
        pltpu.sync_copy(et_hbm.at[pl.ds(base, EB)], et_v)

        def ib(i, _):
            sl = pl.ds(i * 16, 16)
            d16 = dst_v[sl]
            enn = et_v[sl] * NN
            r = i // 2
            o = (i % 2) * 16
            idxd_v[r, pl.ds(o, 16)] = enn + d16
            idxs_v[r, pl.ds(o, 16)] = enn + src_v[sl]
            dstw_v[r, pl.ds(o, 16)] = d16
            return 0

        lax.fori_loop(0, EB // 16, ib, 0)

        g_pend0 = _g_fire(0, 0)
        g_pend1 = _g_fire(1, 1)
        qc = []
        for c in range(NIR):
            qc.append(pltpu.async_copy(
                sq_hbm.at[idxd_v.at[c]], e_v.at[pl.ds(c * RB, RB)], sem_q))
            qc.append(pltpu.async_copy(
                sk_hbm.at[idxs_v.at[c]], val_s.at[pl.ds(c * RB, RB)], sem_q))
        for cp in qc:
            cp.wait()

        def lg(i, _):
            sl = pl.ds(i * 16, 16)
            l = e_v[sl] + val_s[sl]
            l = jnp.maximum(l, 0.2 * l)
            e_v[sl] = jnp.exp(l)
            return 0

        lax.fori_loop(0, EB // 16, lg, 0)

        sd = []
        for c in range(NIR):
            sl = pl.ds(c * RB, RB)
            sd.append(pltpu.async_copy(
                e_v.at[sl], s_sh.at[dstw_v.at[c]], sem_sd, add=True))
            if with_deg:
                sd.append(pltpu.async_copy(
                    ones_v, deg_sh.at[dstw_v.at[c]], sem_sd, add=True))

        def quad_body(t, _):
            for j in range(NRING):
                r = t * NRING + j
                _g_wait(r, j)
                j2 = (j + 2) % NRING

                @pl.when(r + 2 < NIR)
                def _():
                    @pl.when(r >= 2)
                    def _():
                        _s_wait(r - 2, j2)
                    _g_fire(r + 2, j2)

                _scale(rows[j], r * RB)
                _s_fire(r, j)
            return 0

        lax.fori_loop(0, NIR // NRING, quad_body, 0)

        _s_wait(NIR - 2, (NIR - 2) % NRING)
        _s_wait(NIR - 1, (NIR - 1) % NRING)
        for cp in sd:
            cp.wait()
        return 0

    lax.fori_loop(0, NSB, sb_body, 0)
    plsc.subcore_barrier()

    sl = pl.ds(sid * NPT, NPT)
    pltpu.sync_copy(s_sh.at[sl], s_out.at[cid, sl])
    if with_deg:
        pltpu.sync_copy(deg_sh.at[sl], deg_out.at[cid, sl])
    pltpu.sync_copy(out_sh.at[sl, :], out_hbm.at[cid, sl, :])


def _make_edge(with_deg=True):
  import functools as _ft
  return pl.kernel(
    _ft.partial(_edge_body, with_deg),
    out_type=[
        jax.ShapeDtypeStruct((NUM_SC, NN), jnp.float32),
        jax.ShapeDtypeStruct((NUM_SC, NN), jnp.float32),
        jax.ShapeDtypeStruct((NUM_SC, NN, DD), jnp.float32),
    ],
    mesh=plsc.VectorSubcoreMesh(core_axis_name="c", subcore_axis_name="s"),
    scratch_types=[
        pltpu.VMEM((EB,), jnp.int32),
        pltpu.VMEM((EB,), jnp.int32),
        pltpu.VMEM((EB,), jnp.int32),
        pltpu.VMEM((NIR, RB), jnp.int32),
        pltpu.VMEM((NIR, RB), jnp.int32),
        pltpu.VMEM((NIR, RB), jnp.int32),
        pltpu.VMEM((EB,), jnp.float32),
        pltpu.VMEM((EB,), jnp.float32),
        pltpu.VMEM((RB,), jnp.float32),
        pltpu.VMEM((RB, DD), jnp.float32),
        pltpu.VMEM((RB, DD), jnp.float32),
        pltpu.VMEM((RB, DD), jnp.float32),
        pltpu.VMEM((RB, DD), jnp.float32),
        pltpu.VMEM_SHARED((NN,), jnp.float32),
        pltpu.VMEM_SHARED((NN,), jnp.float32),
        pltpu.VMEM_SHARED((NN, DD), jnp.float32),
        pltpu.SemaphoreType.DMA,
        pltpu.SemaphoreType.DMA,
        pltpu.SemaphoreType.DMA,
        pltpu.SemaphoreType.DMA,
        pltpu.SemaphoreType.DMA,
        pltpu.SemaphoreType.DMA,
        pltpu.SemaphoreType.DMA,
        pltpu.SemaphoreType.DMA,
        pltpu.SemaphoreType.DMA,
        pltpu.SemaphoreType.DMA,
        pltpu.SemaphoreType.DMA,
    ],
  )


def kernel(
    x0, x1, x2, Pw0, Pb0, Pw1, Pb1, Pw2, Pb2,
    W1, q1, k1, b1, W2, q2, k2, b2,
    S1w, S1b, L1w, L1b, L2w, L2b, edge_index, edge_type,
):
    src = edge_index[0]
    dst = edge_index[1]
    et = edge_type

    l2w_pad = jnp.pad(L2w, ((0, 0), (0, DD - L2w.shape[1])))
    l2b_pad = jnp.pad(L2b, (0, DD - L2b.shape[0]))

    pb_stack = jnp.stack([Pb0, Pb1, Pb2]).reshape(NUM_OMICS, 1, DD)
    xh3, sl3 = _proj3(x0, x1, x2, Pw0, Pw1, Pw2, pb_stack, S1w, S1b)
    xh = xh3.reshape(NN, DD)
    sl = sl3.reshape(NN, DD)

    edge_call = _make_edge(True)
    edge_call2 = _make_edge(False)

    hall1, sq1, sk1 = _hall_scores(xh, W1, q1, k1)
    s1p, degp, op1 = edge_call(
        sq1.reshape(RR * NN), sk1.reshape(RR * NN),
        hall1.reshape(RR * NN, DD), src, dst, et,
    )
    hall2, sq2, sk2 = _hall_scores_combined(op1, s1p, degp, b1, sl, W2, q2, k2)
    s2p, _deg2, op2 = edge_call2(
        sq2.reshape(RR * NN), sk2.reshape(RR * NN),
        hall2.reshape(RR * NN, DD), src, dst, et,
    )

    y = _head(op2, s2p, degp, b2, sl, L1w, L1b, l2w_pad, l2b_pad)
    return y[:, : L2w.shape[1]]


# speedup vs baseline: 1.0192x; 1.0012x over previous
import jax
import jax.numpy as jnp
from jax import lax
from jax.experimental import pallas as pl
from jax.experimental.pallas import tpu as pltpu
from jax.experimental.pallas import tpu_sc as plsc

N0 = 4096
NUM_OMICS = 3
NN = N0 * NUM_OMICS
EE = 393216
RR = 6
DD = 128

NUM_SC = 2
NUM_TILES = 16
NW = NUM_SC * NUM_TILES
TILE_E = EE // NW

NPT = NN // NUM_TILES


def _elu(x):
    return jnp.where(x > 0, x, jnp.exp(jnp.minimum(x, 0.0)) - 1.0)


def _proj3_body(
    x0_ref, x1_ref, x2_ref, pw0_ref, pw1_ref, pw2_ref, pb_ref,
    s1w_ref, s1b_ref, xh_ref, sl_ref,
):
    for i, (x_ref, pw_ref) in enumerate(
        ((x0_ref, pw0_ref), (x1_ref, pw1_ref), (x2_ref, pw2_ref))
    ):
        p = jnp.dot(x_ref[...], pw_ref[...], preferred_element_type=jnp.float32)
        xh = _elu(p + pb_ref[i])
        xh_ref[i] = xh
        sl_ref[i] = (
            jnp.dot(xh, s1w_ref[i], preferred_element_type=jnp.float32)
            + s1b_ref[i]
        )


def _proj3(x0, x1, x2, pw0, pw1, pw2, pb_stack, s1w, s1b):
    bn = 1024
    return pl.pallas_call(
        _proj3_body,
        grid=(N0 // bn,),
        in_specs=[
            pl.BlockSpec((bn, x0.shape[1]), lambda b: (b, 0)),
            pl.BlockSpec((bn, x1.shape[1]), lambda b: (b, 0)),
            pl.BlockSpec((bn, x2.shape[1]), lambda b: (b, 0)),
            pl.BlockSpec((x0.shape[1], DD), lambda b: (0, 0)),
            pl.BlockSpec((x1.shape[1], DD), lambda b: (0, 0)),
            pl.BlockSpec((x2.shape[1], DD), lambda b: (0, 0)),
            pl.BlockSpec((NUM_OMICS, 1, DD), lambda b: (0, 0, 0)),
            pl.BlockSpec((NUM_OMICS, DD, DD), lambda b: (0, 0, 0)),
            pl.BlockSpec((NUM_OMICS, 1, DD), lambda b: (0, 0, 0)),
        ],
        out_specs=[
            pl.BlockSpec((NUM_OMICS, bn, DD), lambda b: (0, b, 0)),
            pl.BlockSpec((NUM_OMICS, bn, DD), lambda b: (0, b, 0)),
        ],
        out_shape=[
            jax.ShapeDtypeStruct((NUM_OMICS, N0, DD), jnp.float32),
            jax.ShapeDtypeStruct((NUM_OMICS, N0, DD), jnp.float32),
        ],
    )(x0, x1, x2, pw0, pw1, pw2, pb_stack, s1w, s1b.reshape(NUM_OMICS, 1, DD))


def _hall_body(x_ref, w_ref, q_ref, k_ref, h_ref, sq_ref, sk_ref):
    x = x_ref[...]
    for r in range(RR):
        h = jnp.dot(x, w_ref[r], preferred_element_type=jnp.float32)
        h_ref[r] = h
        sq_ref[r] = jnp.sum(h * q_ref[r], axis=1, keepdims=True)
        sk_ref[r] = jnp.sum(h * k_ref[r], axis=1, keepdims=True)


def _hall_scores(x, w, q, k):
    bn = 1024
    return pl.pallas_call(
        _hall_body,
        grid=(NN // bn,),
        in_specs=[
            pl.BlockSpec((bn, DD), lambda b: (b, 0)),
            pl.BlockSpec((RR, DD, DD), lambda b: (0, 0, 0)),
            pl.BlockSpec((RR, 1, DD), lambda b: (0, 0, 0)),
            pl.BlockSpec((RR, 1, DD), lambda b: (0, 0, 0)),
        ],
        out_specs=[
            pl.BlockSpec((RR, bn, DD), lambda b: (0, b, 0)),
            pl.BlockSpec((RR, bn, 1), lambda b: (0, b, 0)),
            pl.BlockSpec((RR, bn, 1), lambda b: (0, b, 0)),
        ],
        out_shape=[
            jax.ShapeDtypeStruct((RR, NN, DD), jnp.float32),
            jax.ShapeDtypeStruct((RR, NN, 1), jnp.float32),
            jax.ShapeDtypeStruct((RR, NN, 1), jnp.float32),
        ],
    )(x, w, q.reshape(RR, 1, DD), k.reshape(RR, 1, DD))


def _hallc_body(
    op_ref, sp_ref, dp_ref, b_ref, sl_ref, w_ref, q_ref, k_ref,
    h_ref, sq_ref, sk_ref,
):
    s = sp_ref[0] + sp_ref[1]
    deg = dp_ref[0] + dp_ref[1]
    wn = deg / (s + 1e-16)
    x = _elu(wn * (op_ref[0] + op_ref[1]) + b_ref[...] + sl_ref[...])
    for r in range(RR):
        h = jnp.dot(x, w_ref[r], preferred_element_type=jnp.float32)
        h_ref[r] = h
        sq_ref[r] = jnp.sum(h * q_ref[r], axis=1, keepdims=True)
        sk_ref[r] = jnp.sum(h * k_ref[r], axis=1, keepdims=True)


def _hall_scores_combined(outp, s_part, deg_part, b, sl, w, q, k):
    bn = 1024
    return pl.pallas_call(
        _hallc_body,
        grid=(NN // bn,),
        in_specs=[
            pl.BlockSpec((2, bn, DD), lambda b_: (0, b_, 0)),
            pl.BlockSpec((2, bn, 1), lambda b_: (0, b_, 0)),
            pl.BlockSpec((2, bn, 1), lambda b_: (0, b_, 0)),
            pl.BlockSpec((1, DD), lambda b_: (0, 0)),
            pl.BlockSpec((bn, DD), lambda b_: (b_, 0)),
            pl.BlockSpec((RR, DD, DD), lambda b_: (0, 0, 0)),
            pl.BlockSpec((RR, 1, DD), lambda b_: (0, 0, 0)),
            pl.BlockSpec((RR, 1, DD), lambda b_: (0, 0, 0)),
        ],
        out_specs=[
            pl.BlockSpec((RR, bn, DD), lambda b_: (0, b_, 0)),
            pl.BlockSpec((RR, bn, 1), lambda b_: (0, b_, 0)),
            pl.BlockSpec((RR, bn, 1), lambda b_: (0, b_, 0)),
        ],
        out_shape=[
            jax.ShapeDtypeStruct((RR, NN, DD), jnp.float32),
            jax.ShapeDtypeStruct((RR, NN, 1), jnp.float32),
            jax.ShapeDtypeStruct((RR, NN, 1), jnp.float32),
        ],
    )(
        outp,
        s_part.reshape(2, NN, 1),
        deg_part.reshape(2, NN, 1),
        b.reshape(1, DD),
        sl,
        w,
        q.reshape(RR, 1, DD),
        k.reshape(RR, 1, DD),
    )


def _head_body(
    op_ref, sp_ref, dp_ref, b_ref, sl_ref, l1w_ref, l1b_ref, l2w_ref, l2b_ref, y_ref
):
    s = sp_ref[0] + sp_ref[1]
    deg = dp_ref[0] + dp_ref[1]
    w = deg / (s + 1e-16)
    h = _elu(w * (op_ref[0] + op_ref[1]) + b_ref[...] + sl_ref[...])
    z = _elu(jnp.dot(h, l1w_ref[...], preferred_element_type=jnp.float32) + l1b_ref[...])
    y_ref[...] = (
        jnp.dot(z, l2w_ref[...], preferred_element_type=jnp.float32) + l2b_ref[...]
    )


def _head(outp, s_part, deg_part, b, sl, l1w, l1b, l2w_pad, l2b_pad):
    bn = 1024
    return pl.pallas_call(
        _head_body,
        grid=(N0 // bn,),
        in_specs=[
            pl.BlockSpec((2, bn, DD), lambda b_: (0, b_, 0)),
            pl.BlockSpec((2, bn, 1), lambda b_: (0, b_, 0)),
            pl.BlockSpec((2, bn, 1), lambda b_: (0, b_, 0)),
            pl.BlockSpec((1, DD), lambda b_: (0, 0)),
            pl.BlockSpec((bn, DD), lambda b_: (b_, 0)),
            pl.BlockSpec((DD, DD), lambda b_: (0, 0)),
            pl.BlockSpec((1, DD), lambda b_: (0, 0)),
            pl.BlockSpec((DD, DD), lambda b_: (0, 0)),
            pl.BlockSpec((1, DD), lambda b_: (0, 0)),
        ],
        out_specs=pl.BlockSpec((bn, DD), lambda b_: (b_, 0)),
        out_shape=jax.ShapeDtypeStruct((N0, DD), jnp.float32),
    )(
        outp,
        s_part.reshape(2, NN, 1),
        deg_part.reshape(2, NN, 1),
        b.reshape(1, DD),
        sl,
        l1w,
        l1b.reshape(1, DD),
        l2w_pad,
        l2b_pad.reshape(1, DD),
    )


EB = 768
RB = 32
NIR = EB // RB
NSB = TILE_E // EB
NRING = 4


def _zero_fill(buf, nwords):
    z = jnp.zeros((16,), jnp.float32)

    def body(i, _):
        buf[pl.ds(i * 16, 16)] = z
        return 0

    lax.fori_loop(0, nwords // 16, body, 0)


def _edge_body(
    with_deg,
    sq_hbm, sk_hbm, hall_hbm, src_hbm, dst_hbm, et_hbm,
    s_out, deg_out, out_hbm,
    src_v, dst_v, et_v, idxd_v, idxs_v, dstw_v,
    val_s, e_v, ones_v, r0_v, r1_v, r2_v, r3_v,
    s_sh, deg_sh, out_sh,
    sem_q, sem_sd, sg0, sg1, sg2, sg3, ss0, ss1, ss2, ss3, sem_z,
):
    cid = lax.axis_index("c")
    sid = lax.axis_index("s")
    base0 = (cid * NUM_TILES + sid) * TILE_E
    rows = (r0_v, r1_v, r2_v, r3_v)
    sgs = (sg0, sg1, sg2, sg3)
    sss = (ss0, ss1, ss2, ss3)

    def zf(i, _):
        r0_v[i // 8, pl.ds((i % 8) * 16, 16)] = jnp.zeros((16,), jnp.float32)
        return 0

    lax.fori_loop(0, RB * 8, zf, 0)
    _zero_fill(e_v, EB)

    if with_deg:
        def fo(i, _):
            ones_v[pl.ds(i * 16, 16)] = jnp.ones((16,), jnp.float32)
            return 0

        lax.fori_loop(0, RB // 16, fo, 0)

    zc = [
        pltpu.async_copy(
            r0_v, out_sh.at[pl.ds(sid * NPT + i * RB, RB), :], sem_z
        )
        for i in range(NPT // RB)
    ]
    for cp in zc:
        cp.wait()
    pltpu.sync_copy(e_v.at[pl.ds(0, NPT)], s_sh.at[pl.ds(sid * NPT, NPT)])
    if with_deg:
        pltpu.sync_copy(e_v.at[pl.ds(0, NPT)], deg_sh.at[pl.ds(sid * NPT, NPT)])
    plsc.subcore_barrier()

    def _scale(rbuf, eoff):
        def sc(g, _):
            cf16 = e_v[pl.ds(eoff + g * 16, 16)]
            for jj in range(16):
                cf = cf16[jj]
                for j in range(DD // 16):
                    sl2 = pl.ds(j * 16, 16)
                    rbuf[g * 16 + jj, sl2] = rbuf[g * 16 + jj, sl2] * cf
            return 0

        lax.fori_loop(0, RB // 16, sc, 0)

    def _g_fire(r, j):
        return pltpu.async_copy(hall_hbm.at[idxs_v.at[r]], rows[j], sgs[j])

    def _g_wait(r, j):
        pltpu.make_async_copy(hall_hbm.at[idxs_v.at[r]], rows[j], sgs[j]).wait()

    def _s_fire(r, j):
        return pltpu.async_copy(rows[j], out_sh.at[dstw_v.at[r]], sss[j], add=True)

    def _s_wait(r, j):
        pltpu.make_async_copy(rows[j], out_sh.at[dstw_v.at[r]], sss[j]).wait()

    def sb_body(sb, _):
        base = base0 + sb * EB
        pltpu.sync_copy(src_hbm.at[pl.ds(base, EB)], src_v)
        pltpu.sync_copy(dst_hbm.at[pl.ds(base, EB)], dst_v)
        pltpu.sync_copy(et_hbm.at[pl.ds(base, EB)], et_v)

        def ib(i, _):
            sl = pl.ds(i * 16, 16)
            d16 = dst_v[sl]
            enn = et_v[sl] * NN
            r = i // 2
            o = (i % 2) * 16
            idxd_v[r, pl.ds(o, 16)] = enn + d16
            idxs_v[r, pl.ds(o, 16)] = enn + src_v[sl]
            dstw_v[r, pl.ds(o, 16)] = d16
            return 0

        lax.fori_loop(0, EB // 16, ib, 0)

        g_pend0 = _g_fire(0, 0)
        g_pend1 = _g_fire(1, 1)
        qc = []
        for c in range(NIR):
            qc.append(pltpu.async_copy(
                sq_hbm.at[idxd_v.at[c]], e_v.at[pl.ds(c * RB, RB)], sem_q))
            qc.append(pltpu.async_copy(
                sk_hbm.at[idxs_v.at[c]], val_s.at[pl.ds(c * RB, RB)], sem_q))
        for cp in qc:
            cp.wait()

        def lg(i, _):
            sl = pl.ds(i * 16, 16)
            l = e_v[sl] + val_s[sl]
            l = jnp.maximum(l, 0.2 * l)
            e_v[sl] = jnp.exp(l)
            return 0

        lax.fori_loop(0, EB // 16, lg, 0)

        sd = []
        for c in range(NIR):
            sl = pl.ds(c * RB, RB)
            sd.append(pltpu.async_copy(
                e_v.at[sl], s_sh.at[dstw_v.at[c]], sem_sd, add=True))
            if with_deg:
                sd.append(pltpu.async_copy(
                    ones_v, deg_sh.at[dstw_v.at[c]], sem_sd, add=True))

        def quad_body(t, _):
            for j in range(NRING):
                r = t * NRING + j
                _g_wait(r, j)
                j2 = (j + 2) % NRING

                @pl.when(r + 2 < NIR)
                def _():
                    @pl.when(r >= 2)
                    def _():
                        _s_wait(r - 2, j2)
                    _g_fire(r + 2, j2)

                _scale(rows[j], r * RB)
                _s_fire(r, j)
            return 0

        lax.fori_loop(0, NIR // NRING, quad_body, 0)

        _s_wait(NIR - 2, (NIR - 2) % NRING)
        _s_wait(NIR - 1, (NIR - 1) % NRING)
        for cp in sd:
            cp.wait()
        return 0

    lax.fori_loop(0, NSB, sb_body, 0)
    plsc.subcore_barrier()

    sl = pl.ds(sid * NPT, NPT)
    pltpu.sync_copy(s_sh.at[sl], s_out.at[cid, sl])
    if with_deg:
        pltpu.sync_copy(deg_sh.at[sl], deg_out.at[cid, sl])
    pltpu.sync_copy(out_sh.at[sl, :], out_hbm.at[cid, sl, :])


def _make_edge(with_deg=True):
  import functools as _ft
  return pl.kernel(
    _ft.partial(_edge_body, with_deg),
    out_type=[
        jax.ShapeDtypeStruct((NUM_SC, NN), jnp.float32),
        jax.ShapeDtypeStruct((NUM_SC, NN), jnp.float32),
        jax.ShapeDtypeStruct((NUM_SC, NN, DD), jnp.float32),
    ],
    mesh=plsc.VectorSubcoreMesh(core_axis_name="c", subcore_axis_name="s"),
    scratch_types=[
        pltpu.VMEM((EB,), jnp.int32),
        pltpu.VMEM((EB,), jnp.int32),
        pltpu.VMEM((EB,), jnp.int32),
        pltpu.VMEM((NIR, RB), jnp.int32),
        pltpu.VMEM((NIR, RB), jnp.int32),
        pltpu.VMEM((NIR, RB), jnp.int32),
        pltpu.VMEM((EB,), jnp.float32),
        pltpu.VMEM((EB,), jnp.float32),
        pltpu.VMEM((RB,), jnp.float32),
        pltpu.VMEM((RB, DD), jnp.float32),
        pltpu.VMEM((RB, DD), jnp.float32),
        pltpu.VMEM((RB, DD), jnp.float32),
        pltpu.VMEM((RB, DD), jnp.float32),
        pltpu.VMEM_SHARED((NN,), jnp.float32),
        pltpu.VMEM_SHARED((NN,), jnp.float32),
        pltpu.VMEM_SHARED((NN, DD), jnp.float32),
        pltpu.SemaphoreType.DMA,
        pltpu.SemaphoreType.DMA,
        pltpu.SemaphoreType.DMA,
        pltpu.SemaphoreType.DMA,
        pltpu.SemaphoreType.DMA,
        pltpu.SemaphoreType.DMA,
        pltpu.SemaphoreType.DMA,
        pltpu.SemaphoreType.DMA,
        pltpu.SemaphoreType.DMA,
        pltpu.SemaphoreType.DMA,
        pltpu.SemaphoreType.DMA,
    ],
  )


def kernel(
    x0, x1, x2, Pw0, Pb0, Pw1, Pb1, Pw2, Pb2,
    W1, q1, k1, b1, W2, q2, k2, b2,
    S1w, S1b, L1w, L1b, L2w, L2b, edge_index, edge_type,
):
    src = edge_index[0]
    dst = edge_index[1]
    et = edge_type

    l2w_pad = jnp.pad(L2w, ((0, 0), (0, DD - L2w.shape[1])))
    l2b_pad = jnp.pad(L2b, (0, DD - L2b.shape[0]))

    pb_stack = jnp.stack([Pb0, Pb1, Pb2]).reshape(NUM_OMICS, 1, DD)
    xh3, sl3 = _proj3(x0, x1, x2, Pw0, Pw1, Pw2, pb_stack, S1w, S1b)
    xh = xh3.reshape(NN, DD)
    sl = sl3.reshape(NN, DD)

    edge_call = _make_edge(True)
    edge_call2 = _make_edge(False)

    hall1, sq1, sk1 = _hall_scores(xh, W1, q1, k1)
    s1p, degp, op1 = edge_call(
        sq1.reshape(RR * NN), sk1.reshape(RR * NN),
        hall1.reshape(RR * NN, DD), src, dst, et,
    )
    hall2, sq2, sk2 = _hall_scores_combined(op1, s1p, degp, b1, sl, W2, q2, k2)
    s2p, _deg2, op2 = edge_call2(
        sq2.reshape(RR * NN), sk2.reshape(RR * NN),
        hall2.reshape(RR * NN, DD), src, dst, et,
    )

    y = _head(op2, s2p, degp, b2, sl, L1w, L1b, l2w_pad, l2b_pad)
    return y[:, : L2w.shape[1]]
